# baseline (speedup 1.0000x reference)
import jax
import jax.numpy as jnp
from jax import lax
from jax.experimental import pallas as pl
from jax.experimental.pallas import tpu as pltpu
from jax.experimental.pallas import tpu_sc as plsc

DIM = 128
BATCH = 1024 * 200

NC = 2
NS = 16
NW = NC * NS
B_PER_W = BATCH // NW
CHUNK = 64
N_CHUNKS = B_PER_W // CHUNK
NBUF = 5
NGROUPS = N_CHUNKS // NBUF


def _emb_body(idx_hbm, table_hbm, out_hbm, idx_v, rows_v, gsem, wsem):
    wid = lax.axis_index("s") * NC + lax.axis_index("c")
    base = wid * B_PER_W

    pltpu.sync_copy(idx_hbm.at[pl.ds(base, B_PER_W)], idx_v)

    def gather(c, b):
        return pltpu.make_async_copy(
            table_hbm.at[idx_v.at[pl.ds(c * CHUNK, CHUNK)]],
            rows_v.at[b], gsem.at[b])

    def write(c, b):
        return pltpu.make_async_copy(
            rows_v.at[b], out_hbm.at[pl.ds(base + c * CHUNK, CHUNK)],
            wsem.at[b])

    LAG = 3

    for b in range(LAG):
        gather(b, b).start()
    for b in range(NBUF):
        cn = b + LAG
        if cn < NBUF:
            gather(cn, cn).start()
        else:
            write(cn - NBUF, cn - NBUF).wait()
            gather(cn, cn - NBUF).start()
        gather(b, b).wait()
        write(b, b).start()

    def group(g, carry):
        for b in range(NBUF):
            c = g * NBUF + b
            bn = (b + LAG) % NBUF
            write(c + LAG - NBUF, bn).wait()
            gather(c + LAG, bn).start()
            gather(c, b).wait()
            write(c, b).start()
        return carry

    lax.fori_loop(1, NGROUPS - 1, group, 0)

    last = (NGROUPS - 1) * NBUF
    for b in range(NBUF):
        c = last + b
        cn = c + LAG
        if cn < N_CHUNKS:
            bn = (b + LAG) % NBUF
            write(cn - NBUF, bn).wait()
            gather(cn, bn).start()
        gather(c, b).wait()
        write(c, b).start()
    for b in range(NBUF):
        write(last + b, b).wait()


@jax.jit
def _embedding_sc(idx, table):
    mesh = plsc.VectorSubcoreMesh(core_axis_name="c", subcore_axis_name="s")
    f = pl.kernel(
        _emb_body,
        out_type=jax.ShapeDtypeStruct((BATCH, DIM), jnp.float32),
        mesh=mesh,
        scratch_types=[
            pltpu.VMEM((B_PER_W,), jnp.int32),
            pltpu.VMEM((NBUF, CHUNK, DIM), jnp.float32),
            pltpu.SemaphoreType.DMA((NBUF,)),
            pltpu.SemaphoreType.DMA((NBUF,)),
        ],
    )
    return f(idx, table)


def kernel(x, weight):
    idx = x.reshape(-1).astype(jnp.int32)
    out = _embedding_sc(idx, weight)
    return out.reshape(x.shape + (DIM,))


# speedup vs baseline: 1.0052x; 1.0052x over previous
import jax
import jax.numpy as jnp
from jax import lax
from jax.experimental import pallas as pl
from jax.experimental.pallas import tpu as pltpu
from jax.experimental.pallas import tpu_sc as plsc

DIM = 128
BATCH = 1024 * 200

NC = 2
NS = 16
NW = NC * NS
B_PER_W = BATCH // NW
CHUNK = 128
N_CHUNKS = B_PER_W // CHUNK
NBUF = 5
NGROUPS = N_CHUNKS // NBUF


def _emb_body(idx_hbm, table_hbm, out_hbm, idx_v, rows_v, gsem, wsem):
    wid = lax.axis_index("s") * NC + lax.axis_index("c")
    base = wid * B_PER_W

    pltpu.sync_copy(idx_hbm.at[pl.ds(base, B_PER_W)], idx_v)

    def gather(c, b):
        return pltpu.make_async_copy(
            table_hbm.at[idx_v.at[pl.ds(c * CHUNK, CHUNK)]],
            rows_v.at[b], gsem.at[b])

    def write(c, b):
        return pltpu.make_async_copy(
            rows_v.at[b], out_hbm.at[pl.ds(base + c * CHUNK, CHUNK)],
            wsem.at[b])

    LAG = 3

    for b in range(LAG):
        gather(b, b).start()
    for b in range(NBUF):
        cn = b + LAG
        if cn < NBUF:
            gather(cn, cn).start()
        else:
            write(cn - NBUF, cn - NBUF).wait()
            gather(cn, cn - NBUF).start()
        gather(b, b).wait()
        write(b, b).start()

    def group(g, carry):
        for b in range(NBUF):
            c = g * NBUF + b
            bn = (b + LAG) % NBUF
            write(c + LAG - NBUF, bn).wait()
            gather(c + LAG, bn).start()
            gather(c, b).wait()
            write(c, b).start()
        return carry

    lax.fori_loop(1, NGROUPS - 1, group, 0)

    last = (NGROUPS - 1) * NBUF
    for b in range(NBUF):
        c = last + b
        cn = c + LAG
        if cn < N_CHUNKS:
            bn = (b + LAG) % NBUF
            write(cn - NBUF, bn).wait()
            gather(cn, bn).start()
        gather(c, b).wait()
        write(c, b).start()
    for b in range(NBUF):
        write(last + b, b).wait()


@jax.jit
def _embedding_sc(idx, table):
    mesh = plsc.VectorSubcoreMesh(core_axis_name="c", subcore_axis_name="s")
    f = pl.kernel(
        _emb_body,
        out_type=jax.ShapeDtypeStruct((BATCH, DIM), jnp.float32),
        mesh=mesh,
        scratch_types=[
            pltpu.VMEM((B_PER_W,), jnp.int32),
            pltpu.VMEM((NBUF, CHUNK, DIM), jnp.float32),
            pltpu.SemaphoreType.DMA((NBUF,)),
            pltpu.SemaphoreType.DMA((NBUF,)),
        ],
    )
    return f(idx, table)


def kernel(x, weight):
    idx = x.reshape(-1).astype(jnp.int32)
    out = _embedding_sc(idx, weight)
    return out.reshape(x.shape + (DIM,))
